# baseline (speedup 1.0000x reference)
import functools

import numpy as np

import jax
import jax.numpy as jnp
from jax import lax
from jax.experimental import pallas as pl
from jax.experimental.pallas import tpu as pltpu
from jax.experimental.pallas import tpu_sc as plsc

N_GRAPHS = 256

_CW = 64
_K = 64
_NC = 2
_NS = 16


def _cdiv(a, b):
    return (a + b - 1) // b


def _npad(N):
    return _cdiv(N, _NS * 8) * _NS * 8


def _mm_body(*refs, widths, offs, act, scaled):
    if scaled:
        x_ref, w_ref, b_ref, s_ref, *o_refs = refs
        xv = x_ref[...] / jnp.maximum(s_ref[...], 1.0)
    else:
        x_ref, w_ref, b_ref, *o_refs = refs
        xv = x_ref[...]
    acc = jnp.dot(xv, w_ref[...], preferred_element_type=jnp.float32)
    acc = acc + b_ref[...]
    if act == "leaky":
        acc = jnp.where(acc >= 0, acc, 0.01 * acc)
    for o_ref, off, wd in zip(o_refs, offs, widths):
        o_ref[...] = acc[:, off : off + wd]


def _mm_multi(x, w, b, widths, act=None, seg_count=None, bm=1024):
    M, K = x.shape
    Nout = w.shape[1]
    bm = min(bm, M)
    grid = _cdiv(M, bm)
    offs = [int(o) for o in np.cumsum([0] + list(widths))[:-1]]
    in_specs = [
        pl.BlockSpec((bm, K), lambda i: (i, 0)),
        pl.BlockSpec((K, Nout), lambda i: (0, 0)),
        pl.BlockSpec((1, Nout), lambda i: (0, 0)),
    ]
    args = [x, w, b.reshape(1, Nout)]
    if seg_count is not None:
        in_specs.append(pl.BlockSpec((bm, 1), lambda i: (i, 0)))
        args.append(seg_count)
    out = pl.pallas_call(
        functools.partial(_mm_body, widths=widths, offs=offs, act=act,
                          scaled=seg_count is not None),
        grid=(grid,),
        in_specs=in_specs,
        out_specs=[pl.BlockSpec((bm, wd), lambda i: (i, 0)) for wd in widths],
        out_shape=[jax.ShapeDtypeStruct((M, wd), jnp.float32) for wd in widths],
    )(*args)
    return out


def _mm(x, w, b, act=None, seg_count=None, bm=1024):
    return _mm_multi(x, w, b, [w.shape[1]], act=act, seg_count=seg_count, bm=bm)[0]


@functools.lru_cache(maxsize=None)
def _make_edge_kernel(E, N, d):
    nch = d // _CW
    nblocks = E // _K
    assert E % _K == 0 and d % _CW == 0 and nch % 2 == 0
    npad = _npad(N)
    nr = npad // _NS
    nw = _NC * _NS
    mesh = plsc.VectorSubcoreMesh(core_axis_name="c", subcore_axis_name="s")

    out_type = jax.ShapeDtypeStruct((_NC * nch * npad, 2 * _CW), jnp.float32)
    scratch_types = [
        pltpu.VMEM((_K,), jnp.int32),
        pltpu.VMEM((_K,), jnp.int32),
        pltpu.VMEM((_K,), jnp.int32),
        pltpu.VMEM((_K,), jnp.int32),
        pltpu.VMEM((_K, 2 * _CW), jnp.float32),
        pltpu.VMEM((_K, 2 * _CW), jnp.float32),
        pltpu.VMEM((_K, 2 * _CW), jnp.float32),
        pltpu.VMEM((_K, 2 * _CW), jnp.float32),
        pltpu.VMEM((_K, _CW), jnp.float32),
        pltpu.VMEM((_K, _CW), jnp.float32),
        pltpu.VMEM_SHARED((npad, 2 * _CW), jnp.float32),
        pltpu.SemaphoreType.DMA,
        pltpu.SemaphoreType.DMA,
        pltpu.SemaphoreType.DMA,
        pltpu.SemaphoreType.DMA,
        pltpu.SemaphoreType.DMA,
        pltpu.SemaphoreType.DMA,
    ]
    ntr_max = _cdiv(nblocks, nw)
    npairs = _cdiv(ntr_max, 2)

    @functools.partial(pl.kernel, mesh=mesh, out_type=out_type,
                       scratch_types=scratch_types)
    def edge_kernel(src_h, dst_h, zeros_h, *rest):
        ap_h = rest[0 : nch // 2]
        sq_h = rest[nch // 2 : nch // 2 + nch]
        ec_h = rest[nch // 2 + nch : nch // 2 + 2 * nch]
        out = rest[nch // 2 + 2 * nch]
        (si0, si1, di0, di1, ab0, ab1, sb0, sb1, eb0, eb1, acc_sp,
         sa0, sa1, ss0, ss1, se0, se1) = rest[nch // 2 + 2 * nch + 1 :]
        sidxs = (si0, si1)
        didxs = (di0, di1)
        abs_ = (ab0, ab1)
        sbs = (sb0, sb1)
        ebs = (eb0, eb1)
        sems = ((sa0, ss0, se0), (sa1, ss1, se1))
        c = lax.axis_index("c")
        s = lax.axis_index("s")
        wid = s * _NC + c
        r0 = s * nr

        for q in range(nch):
            p, hq = q // 2, q % 2
            off = hq * _CW

            def valid(j):
                return wid + j * nw < nblocks

            def issue(j, slot):
                @pl.when(valid(j))
                def _():
                    base = (wid + j * nw) * _K
                    pltpu.sync_copy(src_h.at[pl.ds(base, _K)], sidxs[slot])
                    pltpu.sync_copy(dst_h.at[pl.ds(base, _K)], didxs[slot])
                    pltpu.async_copy(ap_h[p].at[didxs[slot]], abs_[slot],
                                     sems[slot][0])
                    pltpu.async_copy(sq_h[q].at[sidxs[slot]], sbs[slot],
                                     sems[slot][1])
                    pltpu.async_copy(ec_h[q].at[pl.ds(base, _K)], ebs[slot],
                                     sems[slot][2])

            def process(j, slot):
                @pl.when(valid(j))
                def _():
                    base = (wid + j * nw) * _K
                    ab, sb, eb = abs_[slot], sbs[slot], ebs[slot]
                    pltpu.make_async_copy(ap_h[p].at[didxs[slot]], ab,
                                          sems[slot][0]).wait()
                    pltpu.make_async_copy(sq_h[q].at[sidxs[slot]], sb,
                                          sems[slot][1]).wait()
                    pltpu.make_async_copy(ec_h[q].at[pl.ds(base, _K)], eb,
                                          sems[slot][2]).wait()

                    def row(r, rc):
                        for cc in range(_CW // 16):
                            sl = pl.ds(off + cc * 16, 16)
                            sl0 = pl.ds(cc * 16, 16)
                            sl1 = pl.ds(_CW + cc * 16, 16)
                            x = ab[r, sl] + sb[r, sl0] + eb[r, sl0]
                            sg = 1.0 / (1.0 + jnp.exp(-x))
                            msg = sg * sb[r, sl1]
                            ab[r, sl1] = sg
                            ab[r, sl0] = msg
                        return rc

                    lax.fori_loop(0, _K, row, 0)
                    pltpu.sync_copy(ab, acc_sp.at[didxs[slot]], add=True)

            pltpu.sync_copy(zeros_h, acc_sp.at[pl.ds(r0, nr)])
            plsc.subcore_barrier()

            issue(0, 0)

            def pair(i, carry):
                j = 2 * i
                issue(j + 1, 1)
                process(j, 0)
                issue(j + 2, 0)
                process(j + 1, 1)
                return carry

            lax.fori_loop(0, npairs, pair, 0)
            plsc.subcore_barrier()
            obase = (c * nch + q) * npad + r0
            pltpu.sync_copy(acc_sp.at[pl.ds(r0, nr)], out.at[pl.ds(obase, nr)])
            plsc.subcore_barrier()

    return edge_kernel


def _fuse_body(*refs, nch, residual):
    if residual:
        uh_ref, acc_ref, h_ref, g_ref, bt_ref, o_ref = refs
    else:
        uh_ref, acc_ref, g_ref, bt_ref, o_ref = refs
    num = jnp.concatenate(
        [acc_ref[0, q, :, 0:_CW] + acc_ref[1, q, :, 0:_CW] for q in range(nch)],
        axis=-1)
    den = jnp.concatenate(
        [acc_ref[0, q, :, _CW:] + acc_ref[1, q, :, _CW:] for q in range(nch)],
        axis=-1)
    hn = uh_ref[...] + num / (den + 1e-6)
    if residual:
        hn = hn + h_ref[...]
    m = jnp.mean(hn, axis=-1, keepdims=True)
    v = jnp.mean((hn - m) * (hn - m), axis=-1, keepdims=True)
    hn = (hn - m) * lax.rsqrt(v + 1e-5) * g_ref[...] + bt_ref[...]
    o_ref[...] = jnp.maximum(hn, 0.0)


def _fuse(uh, acc_p, h_res, g, bt, bn=1000):
    N, d = uh.shape
    nch = d // _CW
    npad = _npad(N)
    grid = _cdiv(N, bn)
    acc4 = acc_p.reshape(_NC, nch, npad, 2 * _CW)
    in_specs = [
        pl.BlockSpec((bn, d), lambda i: (i, 0)),
        pl.BlockSpec((_NC, nch, bn, 2 * _CW), lambda i: (0, 0, i, 0)),
    ]
    args = [uh, acc4]
    if h_res is not None:
        in_specs.append(pl.BlockSpec((bn, d), lambda i: (i, 0)))
        args.append(h_res)
    in_specs += [
        pl.BlockSpec((1, d), lambda i: (0, 0)),
        pl.BlockSpec((1, d), lambda i: (0, 0)),
    ]
    args += [g.reshape(1, d), bt.reshape(1, d)]
    return pl.pallas_call(
        functools.partial(_fuse_body, nch=nch, residual=h_res is not None),
        grid=(grid,),
        in_specs=in_specs,
        out_specs=pl.BlockSpec((bn, d), lambda i: (i, 0)),
        out_shape=jax.ShapeDtypeStruct((N, d), jnp.float32),
    )(*args)


def _pool_body(h_ref, b_ref, s_ref, c_ref):
    i = pl.program_id(0)
    bb = b_ref[0]
    gid = lax.broadcasted_iota(jnp.int32, (N_GRAPHS, bb.shape[1]), 0)
    onehot = (bb == gid).astype(jnp.float32)
    s_blk = jnp.dot(onehot, h_ref[...], preferred_element_type=jnp.float32)
    c_blk = jnp.sum(onehot, axis=1, keepdims=True)

    @pl.when(i == 0)
    def _():
        s_ref[...] = s_blk
        c_ref[...] = c_blk

    @pl.when(i > 0)
    def _():
        s_ref[...] += s_blk
        c_ref[...] += c_blk


def _pool(h, batch, bn=1000):
    N, d = h.shape
    grid = _cdiv(N, bn)
    b3 = batch.reshape(grid, 1, bn).astype(jnp.int32)
    return pl.pallas_call(
        _pool_body,
        grid=(grid,),
        in_specs=[
            pl.BlockSpec((bn, d), lambda i: (i, 0)),
            pl.BlockSpec((1, 1, bn), lambda i: (i, 0, 0)),
        ],
        out_specs=[
            pl.BlockSpec((N_GRAPHS, d), lambda i: (0, 0)),
            pl.BlockSpec((N_GRAPHS, 1), lambda i: (0, 0)),
        ],
        out_shape=[
            jax.ShapeDtypeStruct((N_GRAPHS, d), jnp.float32),
            jax.ShapeDtypeStruct((N_GRAPHS, 1), jnp.float32),
        ],
    )(h, b3)


def _graph_model(x, lap, eattr, edge_index, batch, gp):
    N = x.shape[0]
    E = eattr.shape[0]
    xin = jnp.concatenate([x, lap], axis=1)
    win = jnp.concatenate([gp["Wh"], gp["Wp"]], axis=0)
    h = _mm(xin, win, gp["bh"] + gp["bp"])
    e = _mm(eattr, gp["We"], gp["be"])
    src = edge_index[0].astype(jnp.int32)
    dst = edge_index[1].astype(jnp.int32)
    zeros_stripe = jnp.zeros((_npad(N) // _NS, 2 * _CW), jnp.float32)
    for lp in gp["layers"]:
        d_in, d_out = lp["A"].shape
        nch = d_out // _CW
        sv = []
        for q in range(nch):
            sv.append(lp["B"][:, q * _CW : (q + 1) * _CW])
            sv.append(lp["V"][:, q * _CW : (q + 1) * _CW])
        wpack = jnp.concatenate([lp["A"]] + sv + [lp["U"]], axis=1)
        z = jnp.zeros((2 * d_out,), jnp.float32)
        bpack = jnp.concatenate([lp["be"], z, lp["bh"]])
        widths = [2 * _CW] * (nch // 2) + [2 * _CW] * nch + [d_out]
        outs = _mm_multi(h, wpack, bpack, widths)
        aps = outs[0 : nch // 2]
        sqs = outs[nch // 2 : nch // 2 + nch]
        uh = outs[-1]
        ecps = _mm_multi(e, lp["C"], jnp.zeros((d_out,), jnp.float32),
                         [_CW] * nch)
        edge_k = _make_edge_kernel(E, N, d_out)
        acc_p = edge_k(src, dst, zeros_stripe, *aps, *sqs, *ecps)
        h = _fuse(uh, acc_p, h if d_in == d_out else None,
                  lp["g"], lp["bt"])
    return _pool(h, batch)


def _fc(x, layers, seg_count=None):
    n = len(layers)
    for i, lp in enumerate(layers):
        x = _mm(x, lp["W"], lp["b"], act=None if i == n - 1 else "leaky",
                seg_count=seg_count if i == 0 else None)
    return x


def kernel(drug_x, drug_edge_index, drug_batch, drug_lap_enc, drug_edge_attr, prot_x, prot_edge_index, prot_batch, prot_lap_enc, prot_edge_attr, params):
    sd, cd = _graph_model(drug_x, drug_lap_enc, drug_edge_attr,
                          drug_edge_index, drug_batch, params["drug"])
    sp, cp = _graph_model(prot_x, prot_lap_enc, prot_edge_attr,
                          prot_edge_index, prot_batch, params["prot"])
    xd = _fc(sd, params["drug_fc"], seg_count=cd)
    xp = _fc(sp, params["prot_fc"], seg_count=cp)
    x = jnp.concatenate([xd, xp], axis=1)
    return _fc(x, params["top_fc"])


# speedup vs baseline: 1.0243x; 1.0243x over previous
import functools

import numpy as np

import jax
import jax.numpy as jnp
from jax import lax
from jax.experimental import pallas as pl
from jax.experimental.pallas import tpu as pltpu
from jax.experimental.pallas import tpu_sc as plsc

N_GRAPHS = 256

_CW = 64
_K = 64
_NC = 2
_NS = 16


def _cdiv(a, b):
    return (a + b - 1) // b


def _npad(N):
    return _cdiv(N, _NS * 8) * _NS * 8


def _mm_body(*refs, widths, offs, act, scaled):
    if scaled:
        x_ref, w_ref, b_ref, s_ref, *o_refs = refs
        xv = x_ref[...] / jnp.maximum(s_ref[...], 1.0)
    else:
        x_ref, w_ref, b_ref, *o_refs = refs
        xv = x_ref[...]
    acc = jnp.dot(xv, w_ref[...], preferred_element_type=jnp.float32)
    acc = acc + b_ref[...]
    if act == "leaky":
        acc = jnp.where(acc >= 0, acc, 0.01 * acc)
    for o_ref, off, wd in zip(o_refs, offs, widths):
        o_ref[...] = acc[:, off : off + wd]


def _mm_multi(x, w, b, widths, act=None, seg_count=None, bm=1024):
    M, K = x.shape
    Nout = w.shape[1]
    bm = min(bm, M)
    grid = _cdiv(M, bm)
    offs = [int(o) for o in np.cumsum([0] + list(widths))[:-1]]
    in_specs = [
        pl.BlockSpec((bm, K), lambda i: (i, 0)),
        pl.BlockSpec((K, Nout), lambda i: (0, 0)),
        pl.BlockSpec((1, Nout), lambda i: (0, 0)),
    ]
    args = [x, w, b.reshape(1, Nout)]
    if seg_count is not None:
        in_specs.append(pl.BlockSpec((bm, 1), lambda i: (i, 0)))
        args.append(seg_count)
    out = pl.pallas_call(
        functools.partial(_mm_body, widths=widths, offs=offs, act=act,
                          scaled=seg_count is not None),
        grid=(grid,),
        in_specs=in_specs,
        out_specs=[pl.BlockSpec((bm, wd), lambda i: (i, 0)) for wd in widths],
        out_shape=[jax.ShapeDtypeStruct((M, wd), jnp.float32) for wd in widths],
    )(*args)
    return out


def _mm(x, w, b, act=None, seg_count=None, bm=1024):
    return _mm_multi(x, w, b, [w.shape[1]], act=act, seg_count=seg_count, bm=bm)[0]


@functools.lru_cache(maxsize=None)
def _make_edge_kernel(E, N, d):
    nch = d // _CW
    nblocks = E // _K
    assert E % _K == 0 and d % _CW == 0 and nch % 2 == 0
    npad = _npad(N)
    nr = npad // _NS
    nw = _NC * _NS
    mesh = plsc.VectorSubcoreMesh(core_axis_name="c", subcore_axis_name="s")

    out_type = jax.ShapeDtypeStruct((_NC * nch * npad, 2 * _CW), jnp.float32)
    scratch_types = [
        pltpu.VMEM((_K,), jnp.int32),
        pltpu.VMEM((_K,), jnp.int32),
        pltpu.VMEM((_K,), jnp.int32),
        pltpu.VMEM((_K,), jnp.int32),
        pltpu.VMEM((_K, 2 * _CW), jnp.float32),
        pltpu.VMEM((_K, 2 * _CW), jnp.float32),
        pltpu.VMEM((_K, 2 * _CW), jnp.float32),
        pltpu.VMEM((_K, 2 * _CW), jnp.float32),
        pltpu.VMEM((_K, 2 * _CW), jnp.float32),
        pltpu.VMEM((_K, 2 * _CW), jnp.float32),
        pltpu.VMEM_SHARED((npad, 2 * _CW), jnp.float32),
        pltpu.SemaphoreType.DMA,
        pltpu.SemaphoreType.DMA,
        pltpu.SemaphoreType.DMA,
        pltpu.SemaphoreType.DMA,
        pltpu.SemaphoreType.DMA,
        pltpu.SemaphoreType.DMA,
    ]
    ntr_max = _cdiv(nblocks, nw)
    npairs = _cdiv(ntr_max, 2)

    @functools.partial(pl.kernel, mesh=mesh, out_type=out_type,
                       scratch_types=scratch_types)
    def edge_kernel(src_h, dst_h, zeros_h, *rest):
        ap_h = rest[0 : nch // 2]
        sq_h = rest[nch // 2 : nch // 2 + nch]
        ec_h = rest[nch // 2 + nch : nch // 2 + nch + nch // 2]
        out = rest[2 * nch]
        (si0, si1, di0, di1, ab0, ab1, sb0, sb1, eb0, eb1, acc_sp,
         sa0, sa1, ss0, ss1, se0, se1) = rest[2 * nch + 1 :]
        sidxs = (si0, si1)
        didxs = (di0, di1)
        abs_ = (ab0, ab1)
        sbs = (sb0, sb1)
        ebs = (eb0, eb1)
        sems = ((sa0, ss0, se0), (sa1, ss1, se1))
        c = lax.axis_index("c")
        s = lax.axis_index("s")
        wid = s * _NC + c
        r0 = s * nr

        for q in range(nch):
            p, hq = q // 2, q % 2
            off = hq * _CW

            def valid(j):
                return wid + j * nw < nblocks

            def issue(j, slot):
                @pl.when(valid(j))
                def _():
                    base = (wid + j * nw) * _K
                    pltpu.sync_copy(src_h.at[pl.ds(base, _K)], sidxs[slot])
                    pltpu.sync_copy(dst_h.at[pl.ds(base, _K)], didxs[slot])
                    pltpu.async_copy(ap_h[p].at[didxs[slot]], abs_[slot],
                                     sems[slot][0])
                    pltpu.async_copy(sq_h[q].at[sidxs[slot]], sbs[slot],
                                     sems[slot][1])
                    pltpu.async_copy(ec_h[p].at[pl.ds(base, _K)], ebs[slot],
                                     sems[slot][2])

            def process(j, slot):
                @pl.when(valid(j))
                def _():
                    base = (wid + j * nw) * _K
                    ab, sb, eb = abs_[slot], sbs[slot], ebs[slot]
                    pltpu.make_async_copy(ap_h[p].at[didxs[slot]], ab,
                                          sems[slot][0]).wait()
                    pltpu.make_async_copy(sq_h[q].at[sidxs[slot]], sb,
                                          sems[slot][1]).wait()
                    pltpu.make_async_copy(ec_h[p].at[pl.ds(base, _K)], eb,
                                          sems[slot][2]).wait()

                    def row(r, rc):
                        for cc in range(_CW // 16):
                            sl = pl.ds(off + cc * 16, 16)
                            sl0 = pl.ds(cc * 16, 16)
                            sl1 = pl.ds(_CW + cc * 16, 16)
                            x = ab[r, sl] + sb[r, sl0] + eb[r, sl]
                            sg = 1.0 / (1.0 + jnp.exp(-x))
                            msg = sg * sb[r, sl1]
                            ab[r, sl1] = sg
                            ab[r, sl0] = msg
                        return rc

                    lax.fori_loop(0, _K, row, 0)
                    pltpu.sync_copy(ab, acc_sp.at[didxs[slot]], add=True)

            pltpu.sync_copy(zeros_h, acc_sp.at[pl.ds(r0, nr)])
            plsc.subcore_barrier()

            issue(0, 0)

            def pair(i, carry):
                j = 2 * i
                issue(j + 1, 1)
                process(j, 0)
                issue(j + 2, 0)
                process(j + 1, 1)
                return carry

            lax.fori_loop(0, npairs, pair, 0)
            plsc.subcore_barrier()
            obase = (c * nch + q) * npad + r0
            pltpu.sync_copy(acc_sp.at[pl.ds(r0, nr)], out.at[pl.ds(obase, nr)])
            plsc.subcore_barrier()

    return edge_kernel


def _fuse_body(*refs, nch, residual):
    if residual:
        uh_ref, acc_ref, h_ref, g_ref, bt_ref, o_ref = refs
    else:
        uh_ref, acc_ref, g_ref, bt_ref, o_ref = refs
    num = jnp.concatenate(
        [acc_ref[0, q, :, 0:_CW] + acc_ref[1, q, :, 0:_CW] for q in range(nch)],
        axis=-1)
    den = jnp.concatenate(
        [acc_ref[0, q, :, _CW:] + acc_ref[1, q, :, _CW:] for q in range(nch)],
        axis=-1)
    hn = uh_ref[...] + num / (den + 1e-6)
    if residual:
        hn = hn + h_ref[...]
    m = jnp.mean(hn, axis=-1, keepdims=True)
    v = jnp.mean((hn - m) * (hn - m), axis=-1, keepdims=True)
    hn = (hn - m) * lax.rsqrt(v + 1e-5) * g_ref[...] + bt_ref[...]
    o_ref[...] = jnp.maximum(hn, 0.0)


def _fuse(uh, acc_p, h_res, g, bt, bn=1000):
    N, d = uh.shape
    nch = d // _CW
    npad = _npad(N)
    grid = _cdiv(N, bn)
    acc4 = acc_p.reshape(_NC, nch, npad, 2 * _CW)
    in_specs = [
        pl.BlockSpec((bn, d), lambda i: (i, 0)),
        pl.BlockSpec((_NC, nch, bn, 2 * _CW), lambda i: (0, 0, i, 0)),
    ]
    args = [uh, acc4]
    if h_res is not None:
        in_specs.append(pl.BlockSpec((bn, d), lambda i: (i, 0)))
        args.append(h_res)
    in_specs += [
        pl.BlockSpec((1, d), lambda i: (0, 0)),
        pl.BlockSpec((1, d), lambda i: (0, 0)),
    ]
    args += [g.reshape(1, d), bt.reshape(1, d)]
    return pl.pallas_call(
        functools.partial(_fuse_body, nch=nch, residual=h_res is not None),
        grid=(grid,),
        in_specs=in_specs,
        out_specs=pl.BlockSpec((bn, d), lambda i: (i, 0)),
        out_shape=jax.ShapeDtypeStruct((N, d), jnp.float32),
    )(*args)


def _pool_body(h_ref, b_ref, s_ref, c_ref):
    i = pl.program_id(0)
    bb = b_ref[0]
    gid = lax.broadcasted_iota(jnp.int32, (N_GRAPHS, bb.shape[1]), 0)
    onehot = (bb == gid).astype(jnp.float32)
    s_blk = jnp.dot(onehot, h_ref[...], preferred_element_type=jnp.float32)
    c_blk = jnp.sum(onehot, axis=1, keepdims=True)

    @pl.when(i == 0)
    def _():
        s_ref[...] = s_blk
        c_ref[...] = c_blk

    @pl.when(i > 0)
    def _():
        s_ref[...] += s_blk
        c_ref[...] += c_blk


def _pool(h, batch, bn=1000):
    N, d = h.shape
    grid = _cdiv(N, bn)
    b3 = batch.reshape(grid, 1, bn).astype(jnp.int32)
    return pl.pallas_call(
        _pool_body,
        grid=(grid,),
        in_specs=[
            pl.BlockSpec((bn, d), lambda i: (i, 0)),
            pl.BlockSpec((1, 1, bn), lambda i: (i, 0, 0)),
        ],
        out_specs=[
            pl.BlockSpec((N_GRAPHS, d), lambda i: (0, 0)),
            pl.BlockSpec((N_GRAPHS, 1), lambda i: (0, 0)),
        ],
        out_shape=[
            jax.ShapeDtypeStruct((N_GRAPHS, d), jnp.float32),
            jax.ShapeDtypeStruct((N_GRAPHS, 1), jnp.float32),
        ],
    )(h, b3)


def _graph_model(x, lap, eattr, edge_index, batch, gp):
    N = x.shape[0]
    E = eattr.shape[0]
    xin = jnp.concatenate([x, lap], axis=1)
    win = jnp.concatenate([gp["Wh"], gp["Wp"]], axis=0)
    h = _mm(xin, win, gp["bh"] + gp["bp"])
    e = _mm(eattr, gp["We"], gp["be"])
    src = edge_index[0].astype(jnp.int32)
    dst = edge_index[1].astype(jnp.int32)
    zeros_stripe = jnp.zeros((_npad(N) // _NS, 2 * _CW), jnp.float32)
    for lp in gp["layers"]:
        d_in, d_out = lp["A"].shape
        nch = d_out // _CW
        sv = []
        for q in range(nch):
            sv.append(lp["B"][:, q * _CW : (q + 1) * _CW])
            sv.append(lp["V"][:, q * _CW : (q + 1) * _CW])
        wpack = jnp.concatenate([lp["A"]] + sv + [lp["U"]], axis=1)
        z = jnp.zeros((2 * d_out,), jnp.float32)
        bpack = jnp.concatenate([lp["be"], z, lp["bh"]])
        widths = [2 * _CW] * (nch // 2) + [2 * _CW] * nch + [d_out]
        outs = _mm_multi(h, wpack, bpack, widths)
        aps = outs[0 : nch // 2]
        sqs = outs[nch // 2 : nch // 2 + nch]
        uh = outs[-1]
        ecps = _mm_multi(e, lp["C"], jnp.zeros((d_out,), jnp.float32),
                         [2 * _CW] * (nch // 2))
        edge_k = _make_edge_kernel(E, N, d_out)
        acc_p = edge_k(src, dst, zeros_stripe, *aps, *sqs, *ecps)
        h = _fuse(uh, acc_p, h if d_in == d_out else None,
                  lp["g"], lp["bt"])
    return _pool(h, batch)


def _fc(x, layers, seg_count=None):
    n = len(layers)
    for i, lp in enumerate(layers):
        x = _mm(x, lp["W"], lp["b"], act=None if i == n - 1 else "leaky",
                seg_count=seg_count if i == 0 else None)
    return x


def kernel(drug_x, drug_edge_index, drug_batch, drug_lap_enc, drug_edge_attr, prot_x, prot_edge_index, prot_batch, prot_lap_enc, prot_edge_attr, params):
    sd, cd = _graph_model(drug_x, drug_lap_enc, drug_edge_attr,
                          drug_edge_index, drug_batch, params["drug"])
    sp, cp = _graph_model(prot_x, prot_lap_enc, prot_edge_attr,
                          prot_edge_index, prot_batch, params["prot"])
    xd = _fc(sd, params["drug_fc"], seg_count=cd)
    xp = _fc(sp, params["prot_fc"], seg_count=cp)
    x = jnp.concatenate([xd, xp], axis=1)
    return _fc(x, params["top_fc"])
